# baseline (speedup 1.0000x reference)
import functools

import jax
import jax.numpy as jnp
from jax import lax
from jax.experimental import pallas as pl
from jax.experimental.pallas import tpu as pltpu
from jax.experimental.pallas import tpu_sc as plsc

N_NODES = 10000
N_EDGES = 320000
D = 128

NC = 2
NS = 16
NW = NC * NS
N_PAD = 10240
RPT = N_PAD // NS
EPT = N_EDGES // NW
CH = 80
NCHUNK = EPT // CH

_mesh = plsc.VectorSubcoreMesh(core_axis_name="c", subcore_axis_name="s")


def _leaky(x):
    return jnp.where(x >= 0, x, 0.01 * x)


def _sc_degree_body(dst_hbm, zeros_hbm, out_hbm, acc, idxb0, idxb1, onesb,
                    ssem0, ssem1):
    c = lax.axis_index("c")
    s = lax.axis_index("s")
    wid = c * NS + s
    r0 = s * RPT
    pltpu.sync_copy(zeros_hbm.at[pl.ds(r0, RPT)], acc.at[pl.ds(r0, RPT)])
    for i in range(CH // 16):
        onesb[pl.ds(i * 16, 16)] = jnp.ones((16,), jnp.float32)
    plsc.subcore_barrier()
    base0 = wid * EPT

    idxb = [idxb0, idxb1]
    ssem = [ssem0, ssem1]

    def idx_copy(j, b):
        pltpu.sync_copy(dst_hbm.at[pl.ds(base0 + j * CH, CH)], idxb[b])

    def start_scatter(b):
        pltpu.async_copy(onesb, acc.at[idxb[b]], ssem[b], add=True)

    def drain_scatter(b):
        pltpu.make_async_copy(onesb, acc.at[idxb[b]], ssem[b]).wait()

    idx_copy(0, 0)
    start_scatter(0)

    @pl.loop(0, (NCHUNK - 1) // 2)
    def _pair(o):
        for i in range(2):
            j = 2 * o + 1 + i
            b = (1 + i) % 2
            if i == 0:
                @pl.when(o >= 1)
                def _():
                    drain_scatter(b)
            else:
                drain_scatter(b)
            idx_copy(j, b)
            start_scatter(b)

    drain_scatter(0)
    drain_scatter(1)
    plsc.subcore_barrier()
    pltpu.sync_copy(acc.at[pl.ds(r0, RPT)], out_hbm.at[c, pl.ds(r0, RPT)])


_SC_DEGREE_KW = dict(
    out_type=jax.ShapeDtypeStruct((NC, N_PAD), jnp.float32),
    mesh=_mesh,
    scratch_types=[
        pltpu.VMEM_SHARED((N_PAD,), jnp.float32),
        pltpu.VMEM((CH,), jnp.int32),
        pltpu.VMEM((CH,), jnp.int32),
        pltpu.VMEM((CH,), jnp.float32),
        pltpu.SemaphoreType.DMA,
        pltpu.SemaphoreType.DMA,
    ],
)
_sc_degree = pl.kernel(**_SC_DEGREE_KW)(_sc_degree_body)


def _sc_conv_body(y_hbm, src_hbm, dst_hbm, zeros_hbm, out_hbm,
                  acc, srcb0, srcb1, srcb2, dstb0, dstb1, dstb2,
                  rows0, rows1, rows2, sem0, sem1, sem2):
    c = lax.axis_index("c")
    s = lax.axis_index("s")
    wid = c * NS + s
    r0 = s * RPT

    @pl.when(c == 0)
    def _():
        pltpu.sync_copy(y_hbm.at[pl.ds(r0, RPT)], acc.at[pl.ds(r0, RPT)])

    @pl.when(c != 0)
    def _():
        pltpu.sync_copy(zeros_hbm.at[pl.ds(r0, RPT)], acc.at[pl.ds(r0, RPT)])

    plsc.subcore_barrier()
    base0 = wid * EPT

    srcb = [srcb0, srcb1, srcb2]
    dstb = [dstb0, dstb1, dstb2]
    rows = [rows0, rows1, rows2]
    sem = [sem0, sem1, sem2]

    def idx_copy(j, b):
        base = base0 + j * CH
        pltpu.sync_copy(src_hbm.at[pl.ds(base, CH)], srcb[b])
        pltpu.sync_copy(dst_hbm.at[pl.ds(base, CH)], dstb[b])

    def start_gather(b):
        pltpu.async_copy(y_hbm.at[srcb[b]], rows[b], sem[b])

    def finish_scatter(b):
        pltpu.make_async_copy(y_hbm.at[srcb[b]], rows[b], sem[b]).wait()
        pltpu.sync_copy(rows[b], acc.at[dstb[b]], add=True)

    for j in range(2):
        idx_copy(j, j)
        start_gather(j)

    @pl.loop(0, (NCHUNK - 2) // 3)
    def _triple(o):
        for i in range(3):
            j = 3 * o + 2 + i
            b = (2 + i) % 3
            idx_copy(j, b)
            start_gather(b)
            finish_scatter((b + 1) % 3)

    finish_scatter((NCHUNK - 2) % 3)
    finish_scatter((NCHUNK - 1) % 3)

    plsc.subcore_barrier()
    pltpu.sync_copy(acc.at[pl.ds(r0, RPT)], out_hbm.at[c, pl.ds(r0, RPT)])


_SC_CONV_KW = dict(
    out_type=jax.ShapeDtypeStruct((NC, N_PAD, D), jnp.float32),
    mesh=_mesh,
    scratch_types=[
        pltpu.VMEM_SHARED((N_PAD, D), jnp.float32),
        pltpu.VMEM((CH,), jnp.int32),
        pltpu.VMEM((CH,), jnp.int32),
        pltpu.VMEM((CH,), jnp.int32),
        pltpu.VMEM((CH,), jnp.int32),
        pltpu.VMEM((CH,), jnp.int32),
        pltpu.VMEM((CH,), jnp.int32),
        pltpu.VMEM((CH, D), jnp.float32),
        pltpu.VMEM((CH, D), jnp.float32),
        pltpu.VMEM((CH, D), jnp.float32),
        pltpu.SemaphoreType.DMA,
        pltpu.SemaphoreType.DMA,
        pltpu.SemaphoreType.DMA,
    ],
)
_sc_conv = pl.kernel(**_SC_CONV_KW, name="sc_conv1")(_sc_conv_body)
_sc_conv2 = pl.kernel(**_SC_CONV_KW, name="sc_conv2")(_sc_conv_body)


DW = D // 2


def _make_edge_kernel(part_base, ept, nchunk, n_out, name):
    def body(a_hbm, b_hbm, src_hbm, dst_hbm, outa_hbm, outb_hbm,
             srcb0, srcb1, dstb0, dstb1,
             bufa0, bufa1, bufb0, bufb1,
             gsa0, gsa1, gsb0, gsb1, wsa0, wsa1, wsb0, wsb1):
        c = lax.axis_index("c")
        s = lax.axis_index("s")
        wid = c * NS + s
        base0 = part_base + wid * ept
        out0 = wid * ept

        srcb = [srcb0, srcb1]
        dstb = [dstb0, dstb1]
        bufa = [bufa0, bufa1]
        bufb = [bufb0, bufb1]
        gsa = [gsa0, gsa1]
        gsb = [gsb0, gsb1]
        wsa = [wsa0, wsa1]
        wsb = [wsb0, wsb1]

        def idx_copy(j, b):
            base = base0 + j * CH
            pltpu.sync_copy(src_hbm.at[pl.ds(base, CH)], srcb[b])
            pltpu.sync_copy(dst_hbm.at[pl.ds(base, CH)], dstb[b])

        def start_gathers(b):
            pltpu.async_copy(a_hbm.at[srcb[b]], bufa[b], gsa[b])
            pltpu.async_copy(b_hbm.at[dstb[b]], bufb[b], gsb[b])

        def drain_writes(j, b):
            o = pl.ds(out0 + j * CH, CH)
            pltpu.make_async_copy(bufa[b], outa_hbm.at[o], wsa[b]).wait()
            pltpu.make_async_copy(bufb[b], outb_hbm.at[o], wsb[b]).wait()

        def finish_and_write(j, b):
            pltpu.make_async_copy(a_hbm.at[srcb[b]], bufa[b], gsa[b]).wait()
            pltpu.make_async_copy(b_hbm.at[dstb[b]], bufb[b], gsb[b]).wait()
            o = pl.ds(out0 + j * CH, CH)
            pltpu.async_copy(bufa[b], outa_hbm.at[o], wsa[b])
            pltpu.async_copy(bufb[b], outb_hbm.at[o], wsb[b])

        idx_copy(0, 0)
        start_gathers(0)

        @pl.loop(0, (nchunk - 1) // 2)
        def _pair(o):
            for i in range(2):
                j = 2 * o + 1 + i
                b = (1 + i) % 2
                if i == 0:
                    @pl.when(o >= 1)
                    def _():
                        drain_writes(j - 2, b)
                else:
                    drain_writes(j - 2, b)
                idx_copy(j, b)
                start_gathers(b)
                finish_and_write(j - 1, 1 - b)

        if (nchunk - 1) % 2 == 1:
            j = nchunk - 1
            b = j % 2
            drain_writes(j - 2, b)
            idx_copy(j, b)
            start_gathers(b)
            finish_and_write(j - 1, 1 - b)

        last = nchunk - 1
        bl = last % 2
        finish_and_write(last, bl)
        drain_writes(last - 1, 1 - bl)
        drain_writes(last, bl)

    kw = dict(
        out_type=[
            jax.ShapeDtypeStruct((n_out, DW), jnp.int32),
            jax.ShapeDtypeStruct((n_out, DW), jnp.int32),
        ],
        mesh=_mesh,
        scratch_types=[
            pltpu.VMEM((CH,), jnp.int32),
            pltpu.VMEM((CH,), jnp.int32),
            pltpu.VMEM((CH,), jnp.int32),
            pltpu.VMEM((CH,), jnp.int32),
            pltpu.VMEM((CH, DW), jnp.int32),
            pltpu.VMEM((CH, DW), jnp.int32),
            pltpu.VMEM((CH, DW), jnp.int32),
            pltpu.VMEM((CH, DW), jnp.int32),
            pltpu.SemaphoreType.DMA,
            pltpu.SemaphoreType.DMA,
            pltpu.SemaphoreType.DMA,
            pltpu.SemaphoreType.DMA,
            pltpu.SemaphoreType.DMA,
            pltpu.SemaphoreType.DMA,
            pltpu.SemaphoreType.DMA,
            pltpu.SemaphoreType.DMA,
        ],
        compiler_params=pltpu.CompilerParams(use_tc_tiling_on_sc=False),
    )
    return pl.kernel(**kw, name=name)(body)


E_PART1 = 64 * CH * NW
E_PART2 = N_EDGES - E_PART1
_sc_edge_a = _make_edge_kernel(0, E_PART1 // NW, 64, E_PART1, "sc_edge_a")
_sc_edge_b = _make_edge_kernel(E_PART1, E_PART2 // NW, 61, E_PART2, "sc_edge_b")


def _tc_pre_body(x_ref, w_ref, degp_ref, y_ref, dinv_ref):
    deg = degp_ref[0, :] + degp_ref[1, :] + 1.0
    dinv = lax.rsqrt(deg)[:, None]
    dinv_ref[...] = dinv
    xw = jnp.dot(x_ref[...], w_ref[...], preferred_element_type=jnp.float32)
    y_ref[...] = xw * dinv


def _tc_pre(x, w1, degp):
    blk = 640
    grid = N_PAD // blk
    return pl.pallas_call(
        _tc_pre_body,
        grid=(grid,),
        in_specs=[
            pl.BlockSpec((blk, D), lambda i: (i, 0)),
            pl.BlockSpec((D, D), lambda i: (0, 0)),
            pl.BlockSpec((NC, blk), lambda i: (0, i)),
        ],
        out_specs=[
            pl.BlockSpec((blk, D), lambda i: (i, 0)),
            pl.BlockSpec((blk, 1), lambda i: (i, 0)),
        ],
        out_shape=[
            jax.ShapeDtypeStruct((N_PAD, D), jnp.float32),
            jax.ShapeDtypeStruct((N_PAD, 1), jnp.float32),
        ],
    )(x, w1, degp)


def _tc_mid_body(sp_ref, dinv_ref, b_ref, w_ref, y2_ref):
    s = sp_ref[0] + sp_ref[1]
    dinv = dinv_ref[...]
    h = _leaky(s * dinv + b_ref[...][None, :])
    hw = jnp.dot(h, w_ref[...], preferred_element_type=jnp.float32)
    y2_ref[...] = hw * dinv


def _tc_mid(sp, dinv, b1, w2):
    blk = 640
    grid = N_PAD // blk
    return pl.pallas_call(
        _tc_mid_body,
        grid=(grid,),
        in_specs=[
            pl.BlockSpec((NC, blk, D), lambda i: (0, i, 0)),
            pl.BlockSpec((blk, 1), lambda i: (i, 0)),
            pl.BlockSpec((D,), lambda i: (0,)),
            pl.BlockSpec((D, D), lambda i: (0, 0)),
        ],
        out_specs=pl.BlockSpec((blk, D), lambda i: (i, 0)),
        out_shape=jax.ShapeDtypeStruct((N_PAD, D), jnp.float32),
    )(sp, dinv, b1, w2)


def _tc_post_body(sp_ref, dinv_ref, b_ref, fc1w_ref, fc1b_ref, a_ref, bb_ref):
    s = sp_ref[0] + sp_ref[1]
    dinv = dinv_ref[...]
    h = _leaky(s * dinv + b_ref[...][None, :])
    a_ref[...] = jnp.dot(h, fc1w_ref[: D, :],
                         preferred_element_type=jnp.float32).astype(jnp.bfloat16)
    bb_ref[...] = (jnp.dot(h, fc1w_ref[D:, :],
                           preferred_element_type=jnp.float32)
                   + fc1b_ref[...][None, :]).astype(jnp.bfloat16)


def _tc_post(sp, dinv, b2, fc1w, fc1b):
    blk = 640
    grid = N_PAD // blk
    return pl.pallas_call(
        _tc_post_body,
        grid=(grid,),
        in_specs=[
            pl.BlockSpec((NC, blk, D), lambda i: (0, i, 0)),
            pl.BlockSpec((blk, 1), lambda i: (i, 0)),
            pl.BlockSpec((D,), lambda i: (0,)),
            pl.BlockSpec((2 * D, D), lambda i: (0, 0)),
            pl.BlockSpec((D,), lambda i: (0,)),
        ],
        out_specs=[
            pl.BlockSpec((blk, D), lambda i: (i, 0)),
            pl.BlockSpec((blk, D), lambda i: (i, 0)),
        ],
        out_shape=[
            jax.ShapeDtypeStruct((N_PAD, D), jnp.bfloat16),
            jax.ShapeDtypeStruct((N_PAD, D), jnp.bfloat16),
        ],
    )(sp, dinv, b2, fc1w, fc1b)


def _tc_mlp_body(ea_ref, eb_ref, w2_ref, b2_ref, w3_ref, b3_ref, out_ref):
    e1 = _leaky(ea_ref[...].astype(jnp.float32) + eb_ref[...].astype(jnp.float32))
    e2 = _leaky(jnp.dot(e1, w2_ref[...], preferred_element_type=jnp.float32)
                + b2_ref[...][None, :])
    out_ref[...] = (jnp.dot(e2, w3_ref[...], preferred_element_type=jnp.float32)
                    + b3_ref[...][None, :])


def _tc_mlp(ea, eb, fc2w, fc2b, fc3w, fc3b):
    blk = 2560
    n_e = ea.shape[0]
    grid = n_e // blk
    return pl.pallas_call(
        _tc_mlp_body,
        grid=(grid,),
        in_specs=[
            pl.BlockSpec((blk, D), lambda i: (i, 0)),
            pl.BlockSpec((blk, D), lambda i: (i, 0)),
            pl.BlockSpec((D, 64), lambda i: (0, 0)),
            pl.BlockSpec((64,), lambda i: (0,)),
            pl.BlockSpec((64, 3), lambda i: (0, 0)),
            pl.BlockSpec((3,), lambda i: (0,)),
        ],
        out_specs=pl.BlockSpec((blk, 3), lambda i: (i, 0)),
        out_shape=jax.ShapeDtypeStruct((n_e, 3), jnp.float32),
    )(ea, eb, fc2w, fc2b, fc3w, fc3b)


def kernel(x, edge_index, W1, b1, W2, b2, fc1W, fc1b, fc2W, fc2b, fc3W, fc3b):
    src = edge_index[0].astype(jnp.int32)
    dst = edge_index[1].astype(jnp.int32)
    x_pad = jnp.zeros((N_PAD, D), jnp.float32).at[:N_NODES].set(x)
    zeros_n = jnp.zeros((N_PAD,), jnp.float32)
    zeros_nd = jnp.zeros((N_PAD, D), jnp.float32)

    degp = _sc_degree(dst, zeros_n)
    y1, dinv = _tc_pre(x_pad, W1, degp)

    sp1 = _sc_conv(y1, src, dst, zeros_nd)
    y2 = _tc_mid(sp1, dinv, b1, W2)

    sp2 = _sc_conv2(y2, src, dst, zeros_nd)
    a, b = _tc_post(sp2, dinv, b2, fc1W, fc1b)

    ai = jax.lax.bitcast_convert_type(a.reshape(N_PAD, DW, 2), jnp.int32)
    bi = jax.lax.bitcast_convert_type(b.reshape(N_PAD, DW, 2), jnp.int32)

    def unpack(x):
        n = x.shape[0]
        return jax.lax.bitcast_convert_type(x, jnp.bfloat16).reshape(n, D)

    oa1, ob1 = _sc_edge_a(ai, bi, src, dst)
    out_a = _tc_mlp(unpack(oa1), unpack(ob1), fc2W, fc2b, fc3W, fc3b)
    oa2, ob2 = _sc_edge_b(ai, bi, src, dst)
    out_b = _tc_mlp(unpack(oa2), unpack(ob2), fc2W, fc2b, fc3W, fc3b)
    return jnp.concatenate([out_a, out_b], axis=0)


# speedup vs baseline: 2.7001x; 2.7001x over previous
import functools

import jax
import jax.numpy as jnp
from jax import lax
from jax.experimental import pallas as pl
from jax.experimental.pallas import tpu as pltpu
from jax.experimental.pallas import tpu_sc as plsc

N_NODES = 10000
N_EDGES = 320000
D = 128

NC = 2
NS = 16
NW = NC * NS
N_PAD = 10240
RPT = N_PAD // NS
EPT = N_EDGES // NW
CH = 80
NCHUNK = EPT // CH

_mesh = plsc.VectorSubcoreMesh(core_axis_name="c", subcore_axis_name="s")


def _leaky(x):
    return jnp.where(x >= 0, x, 0.01 * x)


def _sc_degree_body(dst_hbm, zeros_hbm, out_hbm, acc, idxb0, idxb1, onesb,
                    ssem0, ssem1):
    c = lax.axis_index("c")
    s = lax.axis_index("s")
    wid = c * NS + s
    r0 = s * RPT
    pltpu.sync_copy(zeros_hbm.at[pl.ds(r0, RPT)], acc.at[pl.ds(r0, RPT)])
    for i in range(CH // 16):
        onesb[pl.ds(i * 16, 16)] = jnp.ones((16,), jnp.float32)
    plsc.subcore_barrier()
    base0 = wid * EPT

    idxb = [idxb0, idxb1]
    ssem = [ssem0, ssem1]

    def idx_copy(j, b):
        pltpu.sync_copy(dst_hbm.at[pl.ds(base0 + j * CH, CH)], idxb[b])

    def start_scatter(b):
        pltpu.async_copy(onesb, acc.at[idxb[b]], ssem[b], add=True)

    def drain_scatter(b):
        pltpu.make_async_copy(onesb, acc.at[idxb[b]], ssem[b]).wait()

    idx_copy(0, 0)
    start_scatter(0)

    @pl.loop(0, (NCHUNK - 1) // 2)
    def _pair(o):
        for i in range(2):
            j = 2 * o + 1 + i
            b = (1 + i) % 2
            if i == 0:
                @pl.when(o >= 1)
                def _():
                    drain_scatter(b)
            else:
                drain_scatter(b)
            idx_copy(j, b)
            start_scatter(b)

    drain_scatter(0)
    drain_scatter(1)
    plsc.subcore_barrier()
    pltpu.sync_copy(acc.at[pl.ds(r0, RPT)], out_hbm.at[c, pl.ds(r0, RPT)])


_SC_DEGREE_KW = dict(
    out_type=jax.ShapeDtypeStruct((NC, N_PAD), jnp.float32),
    mesh=_mesh,
    scratch_types=[
        pltpu.VMEM_SHARED((N_PAD,), jnp.float32),
        pltpu.VMEM((CH,), jnp.int32),
        pltpu.VMEM((CH,), jnp.int32),
        pltpu.VMEM((CH,), jnp.float32),
        pltpu.SemaphoreType.DMA,
        pltpu.SemaphoreType.DMA,
    ],
)
_sc_degree = pl.kernel(**_SC_DEGREE_KW)(_sc_degree_body)


def _sc_conv_body(y_hbm, src_hbm, dst_hbm, zeros_hbm, out_hbm,
                  acc, srcb0, srcb1, srcb2, dstb0, dstb1, dstb2,
                  rows0, rows1, rows2, sem0, sem1, sem2):
    c = lax.axis_index("c")
    s = lax.axis_index("s")
    wid = c * NS + s
    r0 = s * RPT

    @pl.when(c == 0)
    def _():
        pltpu.sync_copy(y_hbm.at[pl.ds(r0, RPT)], acc.at[pl.ds(r0, RPT)])

    @pl.when(c != 0)
    def _():
        pltpu.sync_copy(zeros_hbm.at[pl.ds(r0, RPT)], acc.at[pl.ds(r0, RPT)])

    plsc.subcore_barrier()
    base0 = wid * EPT

    srcb = [srcb0, srcb1, srcb2]
    dstb = [dstb0, dstb1, dstb2]
    rows = [rows0, rows1, rows2]
    sem = [sem0, sem1, sem2]

    def idx_copy(j, b):
        base = base0 + j * CH
        pltpu.sync_copy(src_hbm.at[pl.ds(base, CH)], srcb[b])
        pltpu.sync_copy(dst_hbm.at[pl.ds(base, CH)], dstb[b])

    def start_gather(b):
        pltpu.async_copy(y_hbm.at[srcb[b]], rows[b], sem[b])

    def finish_scatter(b):
        pltpu.make_async_copy(y_hbm.at[srcb[b]], rows[b], sem[b]).wait()
        pltpu.sync_copy(rows[b], acc.at[dstb[b]], add=True)

    for j in range(2):
        idx_copy(j, j)
        start_gather(j)

    @pl.loop(0, (NCHUNK - 2) // 3)
    def _triple(o):
        for i in range(3):
            j = 3 * o + 2 + i
            b = (2 + i) % 3
            idx_copy(j, b)
            start_gather(b)
            finish_scatter((b + 1) % 3)

    finish_scatter((NCHUNK - 2) % 3)
    finish_scatter((NCHUNK - 1) % 3)

    plsc.subcore_barrier()
    pltpu.sync_copy(acc.at[pl.ds(r0, RPT)], out_hbm.at[c, pl.ds(r0, RPT)])


_SC_CONV_KW = dict(
    out_type=jax.ShapeDtypeStruct((NC, N_PAD, D), jnp.float32),
    mesh=_mesh,
    scratch_types=[
        pltpu.VMEM_SHARED((N_PAD, D), jnp.float32),
        pltpu.VMEM((CH,), jnp.int32),
        pltpu.VMEM((CH,), jnp.int32),
        pltpu.VMEM((CH,), jnp.int32),
        pltpu.VMEM((CH,), jnp.int32),
        pltpu.VMEM((CH,), jnp.int32),
        pltpu.VMEM((CH,), jnp.int32),
        pltpu.VMEM((CH, D), jnp.float32),
        pltpu.VMEM((CH, D), jnp.float32),
        pltpu.VMEM((CH, D), jnp.float32),
        pltpu.SemaphoreType.DMA,
        pltpu.SemaphoreType.DMA,
        pltpu.SemaphoreType.DMA,
    ],
)
_sc_conv = pl.kernel(**_SC_CONV_KW, name="sc_conv1")(_sc_conv_body)
_sc_conv2 = pl.kernel(**_SC_CONV_KW, name="sc_conv2")(_sc_conv_body)


def _make_edge_kernel(part_base, ept, nchunk, n_out, name):
    def body(a_hbm, b_hbm, src_hbm, dst_hbm, out_hbm,
             srcb0, srcb1, dstb0, dstb1,
             bufa0, bufa1, bufb0, bufb1,
             gsa0, gsa1, gsb0, gsb1, ws0, ws1):
        c = lax.axis_index("c")
        s = lax.axis_index("s")
        wid = c * NS + s
        base0 = part_base + wid * ept
        out0 = wid * ept

        srcb = [srcb0, srcb1]
        dstb = [dstb0, dstb1]
        bufa = [bufa0, bufa1]
        bufb = [bufb0, bufb1]
        gsa = [gsa0, gsa1]
        gsb = [gsb0, gsb1]
        ws = [ws0, ws1]

        def idx_copy(j, b):
            base = base0 + j * CH
            pltpu.sync_copy(src_hbm.at[pl.ds(base, CH)], srcb[b])
            pltpu.sync_copy(dst_hbm.at[pl.ds(base, CH)], dstb[b])

        def start_gathers(b):
            pltpu.async_copy(a_hbm.at[srcb[b]], bufa[b], gsa[b])
            pltpu.async_copy(b_hbm.at[dstb[b]], bufb[b], gsb[b])

        def drain_write(j, b):
            pltpu.make_async_copy(
                bufa[b], out_hbm.at[pl.ds(out0 + j * CH, CH)], ws[b]).wait()

        def add_and_write(j, b):
            pltpu.make_async_copy(a_hbm.at[srcb[b]], bufa[b], gsa[b]).wait()
            pltpu.make_async_copy(b_hbm.at[dstb[b]], bufb[b], gsb[b]).wait()

            @pl.loop(0, CH)
            def _row(i):
                for l in range(D // 16):
                    sl = pl.ds(l * 16, 16)
                    bufa[b][i, sl] = bufa[b][i, sl] + bufb[b][i, sl]

            pltpu.async_copy(
                bufa[b], out_hbm.at[pl.ds(out0 + j * CH, CH)], ws[b])

        idx_copy(0, 0)
        start_gathers(0)

        @pl.loop(0, (nchunk - 1) // 2)
        def _pair(o):
            for i in range(2):
                j = 2 * o + 1 + i
                b = (1 + i) % 2
                if i == 0:
                    @pl.when(o >= 1)
                    def _():
                        drain_write(j - 2, b)
                else:
                    drain_write(j - 2, b)
                idx_copy(j, b)
                start_gathers(b)
                add_and_write(j - 1, 1 - b)

        if (nchunk - 1) % 2 == 1:
            j = nchunk - 1
            b = j % 2
            drain_write(j - 2, b)
            idx_copy(j, b)
            start_gathers(b)
            add_and_write(j - 1, 1 - b)

        last = nchunk - 1
        bl = last % 2
        add_and_write(last, bl)
        drain_write(last - 1, 1 - bl)
        drain_write(last, bl)

    kw = dict(
        out_type=jax.ShapeDtypeStruct((n_out, D), jnp.float32),
        mesh=_mesh,
        scratch_types=[
            pltpu.VMEM((CH,), jnp.int32),
            pltpu.VMEM((CH,), jnp.int32),
            pltpu.VMEM((CH,), jnp.int32),
            pltpu.VMEM((CH,), jnp.int32),
            pltpu.VMEM((CH, D), jnp.float32),
            pltpu.VMEM((CH, D), jnp.float32),
            pltpu.VMEM((CH, D), jnp.float32),
            pltpu.VMEM((CH, D), jnp.float32),
            pltpu.SemaphoreType.DMA,
            pltpu.SemaphoreType.DMA,
            pltpu.SemaphoreType.DMA,
            pltpu.SemaphoreType.DMA,
            pltpu.SemaphoreType.DMA,
            pltpu.SemaphoreType.DMA,
        ],
    )
    return pl.kernel(**kw, name=name)(body)


E_PART1 = 64 * CH * NW
E_PART2 = N_EDGES - E_PART1
_sc_edge_a = _make_edge_kernel(0, E_PART1 // NW, 64, E_PART1, "sc_edge_a")
_sc_edge_b = _make_edge_kernel(E_PART1, E_PART2 // NW, 61, E_PART2, "sc_edge_b")


def _tc_pre_body(x_ref, w_ref, degp_ref, y_ref, dinv_ref):
    deg = degp_ref[0, :] + degp_ref[1, :] + 1.0
    dinv = lax.rsqrt(deg)[:, None]
    dinv_ref[...] = dinv
    xw = jnp.dot(x_ref[...], w_ref[...], preferred_element_type=jnp.float32)
    y_ref[...] = xw * dinv


def _tc_pre(x, w1, degp):
    blk = 640
    grid = N_PAD // blk
    return pl.pallas_call(
        _tc_pre_body,
        grid=(grid,),
        in_specs=[
            pl.BlockSpec((blk, D), lambda i: (i, 0)),
            pl.BlockSpec((D, D), lambda i: (0, 0)),
            pl.BlockSpec((NC, blk), lambda i: (0, i)),
        ],
        out_specs=[
            pl.BlockSpec((blk, D), lambda i: (i, 0)),
            pl.BlockSpec((blk, 1), lambda i: (i, 0)),
        ],
        out_shape=[
            jax.ShapeDtypeStruct((N_PAD, D), jnp.float32),
            jax.ShapeDtypeStruct((N_PAD, 1), jnp.float32),
        ],
    )(x, w1, degp)


def _tc_mid_body(sp_ref, dinv_ref, b_ref, w_ref, y2_ref):
    s = sp_ref[0] + sp_ref[1]
    dinv = dinv_ref[...]
    h = _leaky(s * dinv + b_ref[...][None, :])
    hw = jnp.dot(h, w_ref[...], preferred_element_type=jnp.float32)
    y2_ref[...] = hw * dinv


def _tc_mid(sp, dinv, b1, w2):
    blk = 640
    grid = N_PAD // blk
    return pl.pallas_call(
        _tc_mid_body,
        grid=(grid,),
        in_specs=[
            pl.BlockSpec((NC, blk, D), lambda i: (0, i, 0)),
            pl.BlockSpec((blk, 1), lambda i: (i, 0)),
            pl.BlockSpec((D,), lambda i: (0,)),
            pl.BlockSpec((D, D), lambda i: (0, 0)),
        ],
        out_specs=pl.BlockSpec((blk, D), lambda i: (i, 0)),
        out_shape=jax.ShapeDtypeStruct((N_PAD, D), jnp.float32),
    )(sp, dinv, b1, w2)


def _tc_post_body(sp_ref, dinv_ref, b_ref, fc1w_ref, fc1b_ref, a_ref, bb_ref):
    s = sp_ref[0] + sp_ref[1]
    dinv = dinv_ref[...]
    h = _leaky(s * dinv + b_ref[...][None, :])
    a_ref[...] = jnp.dot(h, fc1w_ref[: D, :],
                         preferred_element_type=jnp.float32)
    bb_ref[...] = (jnp.dot(h, fc1w_ref[D:, :],
                           preferred_element_type=jnp.float32)
                   + fc1b_ref[...][None, :])


def _tc_post(sp, dinv, b2, fc1w, fc1b):
    blk = 640
    grid = N_PAD // blk
    return pl.pallas_call(
        _tc_post_body,
        grid=(grid,),
        in_specs=[
            pl.BlockSpec((NC, blk, D), lambda i: (0, i, 0)),
            pl.BlockSpec((blk, 1), lambda i: (i, 0)),
            pl.BlockSpec((D,), lambda i: (0,)),
            pl.BlockSpec((2 * D, D), lambda i: (0, 0)),
            pl.BlockSpec((D,), lambda i: (0,)),
        ],
        out_specs=[
            pl.BlockSpec((blk, D), lambda i: (i, 0)),
            pl.BlockSpec((blk, D), lambda i: (i, 0)),
        ],
        out_shape=[
            jax.ShapeDtypeStruct((N_PAD, D), jnp.float32),
            jax.ShapeDtypeStruct((N_PAD, D), jnp.float32),
        ],
    )(sp, dinv, b2, fc1w, fc1b)


def _tc_mlp_body(e0_ref, w2_ref, b2_ref, w3_ref, b3_ref, out_ref):
    e1 = _leaky(e0_ref[...])
    e2 = _leaky(jnp.dot(e1, w2_ref[...], preferred_element_type=jnp.float32)
                + b2_ref[...][None, :])
    out_ref[...] = (jnp.dot(e2, w3_ref[...], preferred_element_type=jnp.float32)
                    + b3_ref[...][None, :])


def _tc_mlp(e0, fc2w, fc2b, fc3w, fc3b):
    blk = 2560
    n_e = e0.shape[0]
    grid = n_e // blk
    return pl.pallas_call(
        _tc_mlp_body,
        grid=(grid,),
        in_specs=[
            pl.BlockSpec((blk, D), lambda i: (i, 0)),
            pl.BlockSpec((D, 64), lambda i: (0, 0)),
            pl.BlockSpec((64,), lambda i: (0,)),
            pl.BlockSpec((64, 3), lambda i: (0, 0)),
            pl.BlockSpec((3,), lambda i: (0,)),
        ],
        out_specs=pl.BlockSpec((blk, 3), lambda i: (i, 0)),
        out_shape=jax.ShapeDtypeStruct((n_e, 3), jnp.float32),
    )(e0, fc2w, fc2b, fc3w, fc3b)


def kernel(x, edge_index, W1, b1, W2, b2, fc1W, fc1b, fc2W, fc2b, fc3W, fc3b):
    src = edge_index[0].astype(jnp.int32)
    dst = edge_index[1].astype(jnp.int32)
    x_pad = jnp.zeros((N_PAD, D), jnp.float32).at[:N_NODES].set(x)
    zeros_n = jnp.zeros((N_PAD,), jnp.float32)
    zeros_nd = jnp.zeros((N_PAD, D), jnp.float32)

    degp = _sc_degree(dst, zeros_n)
    y1, dinv = _tc_pre(x_pad, W1, degp)

    sp1 = _sc_conv(y1, src, dst, zeros_nd)
    y2 = _tc_mid(sp1, dinv, b1, W2)

    sp2 = _sc_conv2(y2, src, dst, zeros_nd)
    a, b = _tc_post(sp2, dinv, b2, fc1W, fc1b)

    e0a = _sc_edge_a(a, b, src, dst)
    out_a = _tc_mlp(e0a, fc2W, fc2b, fc3W, fc3b)
    e0b = _sc_edge_b(a, b, src, dst)
    out_b = _tc_mlp(e0b, fc2W, fc2b, fc3W, fc3b)
    return jnp.concatenate([out_a, out_b], axis=0)


# speedup vs baseline: 2.7578x; 1.0214x over previous
import functools

import jax
import jax.numpy as jnp
from jax import lax
from jax.experimental import pallas as pl
from jax.experimental.pallas import tpu as pltpu
from jax.experimental.pallas import tpu_sc as plsc

N_NODES = 10000
N_EDGES = 320000
D = 128

NC = 2
NS = 16
NW = NC * NS
N_PAD = 10240
RPT = N_PAD // NS
EPT = N_EDGES // NW
CH = 80
NCHUNK = EPT // CH

_mesh = plsc.VectorSubcoreMesh(core_axis_name="c", subcore_axis_name="s")


def _leaky(x):
    return jnp.where(x >= 0, x, 0.01 * x)


def _sc_degree_body(dst_hbm, zeros_hbm, out_hbm, acc, idxb0, idxb1, onesb,
                    ssem0, ssem1):
    c = lax.axis_index("c")
    s = lax.axis_index("s")
    wid = c * NS + s
    r0 = s * RPT
    pltpu.sync_copy(zeros_hbm.at[pl.ds(r0, RPT)], acc.at[pl.ds(r0, RPT)])
    for i in range(CH // 16):
        onesb[pl.ds(i * 16, 16)] = jnp.ones((16,), jnp.float32)
    plsc.subcore_barrier()
    base0 = wid * EPT

    idxb = [idxb0, idxb1]
    ssem = [ssem0, ssem1]

    def idx_copy(j, b):
        pltpu.sync_copy(dst_hbm.at[pl.ds(base0 + j * CH, CH)], idxb[b])

    def start_scatter(b):
        pltpu.async_copy(onesb, acc.at[idxb[b]], ssem[b], add=True)

    def drain_scatter(b):
        pltpu.make_async_copy(onesb, acc.at[idxb[b]], ssem[b]).wait()

    idx_copy(0, 0)
    start_scatter(0)

    @pl.loop(0, (NCHUNK - 1) // 2)
    def _pair(o):
        for i in range(2):
            j = 2 * o + 1 + i
            b = (1 + i) % 2
            if i == 0:
                @pl.when(o >= 1)
                def _():
                    drain_scatter(b)
            else:
                drain_scatter(b)
            idx_copy(j, b)
            start_scatter(b)

    drain_scatter(0)
    drain_scatter(1)
    plsc.subcore_barrier()
    pltpu.sync_copy(acc.at[pl.ds(r0, RPT)], out_hbm.at[c, pl.ds(r0, RPT)])


_SC_DEGREE_KW = dict(
    out_type=jax.ShapeDtypeStruct((NC, N_PAD), jnp.float32),
    mesh=_mesh,
    scratch_types=[
        pltpu.VMEM_SHARED((N_PAD,), jnp.float32),
        pltpu.VMEM((CH,), jnp.int32),
        pltpu.VMEM((CH,), jnp.int32),
        pltpu.VMEM((CH,), jnp.float32),
        pltpu.SemaphoreType.DMA,
        pltpu.SemaphoreType.DMA,
    ],
)
_sc_degree = pl.kernel(**_SC_DEGREE_KW)(_sc_degree_body)


def _sc_conv_body(y_hbm, src_hbm, dst_hbm, zeros_hbm, out_hbm,
                  acc, srcb0, srcb1, srcb2, dstb0, dstb1, dstb2,
                  rows0, rows1, rows2, sem0, sem1, sem2):
    c = lax.axis_index("c")
    s = lax.axis_index("s")
    wid = c * NS + s
    r0 = s * RPT

    @pl.when(c == 0)
    def _():
        pltpu.sync_copy(y_hbm.at[pl.ds(r0, RPT)], acc.at[pl.ds(r0, RPT)])

    @pl.when(c != 0)
    def _():
        pltpu.sync_copy(zeros_hbm.at[pl.ds(r0, RPT)], acc.at[pl.ds(r0, RPT)])

    plsc.subcore_barrier()
    base0 = wid * EPT

    srcb = [srcb0, srcb1, srcb2]
    dstb = [dstb0, dstb1, dstb2]
    rows = [rows0, rows1, rows2]
    sem = [sem0, sem1, sem2]

    def idx_copy(j, b):
        base = base0 + j * CH
        pltpu.sync_copy(src_hbm.at[pl.ds(base, CH)], srcb[b])
        pltpu.sync_copy(dst_hbm.at[pl.ds(base, CH)], dstb[b])

    def start_gather(b):
        pltpu.async_copy(y_hbm.at[srcb[b]], rows[b], sem[b])

    def finish_scatter(b):
        pltpu.make_async_copy(y_hbm.at[srcb[b]], rows[b], sem[b]).wait()
        pltpu.sync_copy(rows[b], acc.at[dstb[b]], add=True)

    for j in range(2):
        idx_copy(j, j)
        start_gather(j)

    @pl.loop(0, (NCHUNK - 2) // 3)
    def _triple(o):
        for i in range(3):
            j = 3 * o + 2 + i
            b = (2 + i) % 3
            idx_copy(j, b)
            start_gather(b)
            finish_scatter((b + 1) % 3)

    finish_scatter((NCHUNK - 2) % 3)
    finish_scatter((NCHUNK - 1) % 3)

    plsc.subcore_barrier()
    pltpu.sync_copy(acc.at[pl.ds(r0, RPT)], out_hbm.at[c, pl.ds(r0, RPT)])


_SC_CONV_KW = dict(
    out_type=jax.ShapeDtypeStruct((NC, N_PAD, D), jnp.float32),
    mesh=_mesh,
    scratch_types=[
        pltpu.VMEM_SHARED((N_PAD, D), jnp.float32),
        pltpu.VMEM((CH,), jnp.int32),
        pltpu.VMEM((CH,), jnp.int32),
        pltpu.VMEM((CH,), jnp.int32),
        pltpu.VMEM((CH,), jnp.int32),
        pltpu.VMEM((CH,), jnp.int32),
        pltpu.VMEM((CH,), jnp.int32),
        pltpu.VMEM((CH, D), jnp.float32),
        pltpu.VMEM((CH, D), jnp.float32),
        pltpu.VMEM((CH, D), jnp.float32),
        pltpu.SemaphoreType.DMA,
        pltpu.SemaphoreType.DMA,
        pltpu.SemaphoreType.DMA,
    ],
)
_sc_conv = pl.kernel(**_SC_CONV_KW, name="sc_conv1")(_sc_conv_body)
_sc_conv2 = pl.kernel(**_SC_CONV_KW, name="sc_conv2")(_sc_conv_body)


def _make_edge_kernel(part_base, ept, nchunk, n_out, name):
    def body(a_hbm, b_hbm, src_hbm, dst_hbm, out_hbm,
             srcb0, srcb1, dstb0, dstb1,
             bufa0, bufa1, bufb0, bufb1,
             gsa0, gsa1, gsb0, gsb1, ws0, ws1):
        c = lax.axis_index("c")
        s = lax.axis_index("s")
        wid = c * NS + s
        base0 = part_base + wid * ept
        out0 = wid * ept

        srcb = [srcb0, srcb1]
        dstb = [dstb0, dstb1]
        bufa = [bufa0, bufa1]
        bufb = [bufb0, bufb1]
        gsa = [gsa0, gsa1]
        gsb = [gsb0, gsb1]
        ws = [ws0, ws1]

        def idx_copy(j, b):
            base = base0 + j * CH
            pltpu.sync_copy(src_hbm.at[pl.ds(base, CH)], srcb[b])
            pltpu.sync_copy(dst_hbm.at[pl.ds(base, CH)], dstb[b])

        def start_gathers(b):
            pltpu.async_copy(a_hbm.at[srcb[b]], bufa[b], gsa[b])
            pltpu.async_copy(b_hbm.at[dstb[b]], bufb[b], gsb[b])

        def drain_write(j, b):
            pltpu.make_async_copy(
                bufa[b], out_hbm.at[pl.ds(out0 + j * CH, CH)], ws[b]).wait()

        def add_and_write(j, b):
            pltpu.make_async_copy(a_hbm.at[srcb[b]], bufa[b], gsa[b]).wait()
            pltpu.make_async_copy(b_hbm.at[dstb[b]], bufb[b], gsb[b]).wait()

            @pl.loop(0, CH)
            def _row(i):
                for l in range(D // 16):
                    sl = pl.ds(l * 16, 16)
                    bufa[b][i, sl] = bufa[b][i, sl] + bufb[b][i, sl]

            pltpu.async_copy(
                bufa[b], out_hbm.at[pl.ds(out0 + j * CH, CH)], ws[b])

        idx_copy(0, 0)
        start_gathers(0)

        @pl.loop(0, (nchunk - 1) // 2)
        def _pair(o):
            for i in range(2):
                j = 2 * o + 1 + i
                b = (1 + i) % 2
                if i == 0:
                    @pl.when(o >= 1)
                    def _():
                        drain_write(j - 2, b)
                else:
                    drain_write(j - 2, b)
                idx_copy(j, b)
                start_gathers(b)
                add_and_write(j - 1, 1 - b)

        if (nchunk - 1) % 2 == 1:
            j = nchunk - 1
            b = j % 2
            drain_write(j - 2, b)
            idx_copy(j, b)
            start_gathers(b)
            add_and_write(j - 1, 1 - b)

        last = nchunk - 1
        bl = last % 2
        add_and_write(last, bl)
        drain_write(last - 1, 1 - bl)
        drain_write(last, bl)

    kw = dict(
        out_type=jax.ShapeDtypeStruct((n_out, D), jnp.float32),
        mesh=_mesh,
        scratch_types=[
            pltpu.VMEM((CH,), jnp.int32),
            pltpu.VMEM((CH,), jnp.int32),
            pltpu.VMEM((CH,), jnp.int32),
            pltpu.VMEM((CH,), jnp.int32),
            pltpu.VMEM((CH, D), jnp.float32),
            pltpu.VMEM((CH, D), jnp.float32),
            pltpu.VMEM((CH, D), jnp.float32),
            pltpu.VMEM((CH, D), jnp.float32),
            pltpu.SemaphoreType.DMA,
            pltpu.SemaphoreType.DMA,
            pltpu.SemaphoreType.DMA,
            pltpu.SemaphoreType.DMA,
            pltpu.SemaphoreType.DMA,
            pltpu.SemaphoreType.DMA,
        ],
    )
    return pl.kernel(**kw, name=name)(body)


_EDGE_CHUNKS = (32, 31, 31, 31)
_sc_edges = []
_EDGE_SIZES = []
_pbase = 0
for _i, _nch in enumerate(_EDGE_CHUNKS):
    _n = _nch * CH * NW
    _sc_edges.append(
        _make_edge_kernel(_pbase, _nch * CH, _nch, _n, f"sc_edge_{_i}"))
    _EDGE_SIZES.append(_n)
    _pbase += _n


def _tc_pre_body(x_ref, w_ref, degp_ref, y_ref, dinv_ref):
    deg = degp_ref[0, :] + degp_ref[1, :] + 1.0
    dinv = lax.rsqrt(deg)[:, None]
    dinv_ref[...] = dinv
    xw = jnp.dot(x_ref[...], w_ref[...], preferred_element_type=jnp.float32)
    y_ref[...] = xw * dinv


def _tc_pre(x, w1, degp):
    blk = 640
    grid = N_PAD // blk
    return pl.pallas_call(
        _tc_pre_body,
        grid=(grid,),
        in_specs=[
            pl.BlockSpec((blk, D), lambda i: (i, 0)),
            pl.BlockSpec((D, D), lambda i: (0, 0)),
            pl.BlockSpec((NC, blk), lambda i: (0, i)),
        ],
        out_specs=[
            pl.BlockSpec((blk, D), lambda i: (i, 0)),
            pl.BlockSpec((blk, 1), lambda i: (i, 0)),
        ],
        out_shape=[
            jax.ShapeDtypeStruct((N_PAD, D), jnp.float32),
            jax.ShapeDtypeStruct((N_PAD, 1), jnp.float32),
        ],
    )(x, w1, degp)


def _tc_mid_body(sp_ref, dinv_ref, b_ref, w_ref, y2_ref):
    s = sp_ref[0] + sp_ref[1]
    dinv = dinv_ref[...]
    h = _leaky(s * dinv + b_ref[...][None, :])
    hw = jnp.dot(h, w_ref[...], preferred_element_type=jnp.float32)
    y2_ref[...] = hw * dinv


def _tc_mid(sp, dinv, b1, w2):
    blk = 640
    grid = N_PAD // blk
    return pl.pallas_call(
        _tc_mid_body,
        grid=(grid,),
        in_specs=[
            pl.BlockSpec((NC, blk, D), lambda i: (0, i, 0)),
            pl.BlockSpec((blk, 1), lambda i: (i, 0)),
            pl.BlockSpec((D,), lambda i: (0,)),
            pl.BlockSpec((D, D), lambda i: (0, 0)),
        ],
        out_specs=pl.BlockSpec((blk, D), lambda i: (i, 0)),
        out_shape=jax.ShapeDtypeStruct((N_PAD, D), jnp.float32),
    )(sp, dinv, b1, w2)


def _tc_post_body(sp_ref, dinv_ref, b_ref, fc1w_ref, fc1b_ref, a_ref, bb_ref):
    s = sp_ref[0] + sp_ref[1]
    dinv = dinv_ref[...]
    h = _leaky(s * dinv + b_ref[...][None, :])
    a_ref[...] = jnp.dot(h, fc1w_ref[: D, :],
                         preferred_element_type=jnp.float32)
    bb_ref[...] = (jnp.dot(h, fc1w_ref[D:, :],
                           preferred_element_type=jnp.float32)
                   + fc1b_ref[...][None, :])


def _tc_post(sp, dinv, b2, fc1w, fc1b):
    blk = 640
    grid = N_PAD // blk
    return pl.pallas_call(
        _tc_post_body,
        grid=(grid,),
        in_specs=[
            pl.BlockSpec((NC, blk, D), lambda i: (0, i, 0)),
            pl.BlockSpec((blk, 1), lambda i: (i, 0)),
            pl.BlockSpec((D,), lambda i: (0,)),
            pl.BlockSpec((2 * D, D), lambda i: (0, 0)),
            pl.BlockSpec((D,), lambda i: (0,)),
        ],
        out_specs=[
            pl.BlockSpec((blk, D), lambda i: (i, 0)),
            pl.BlockSpec((blk, D), lambda i: (i, 0)),
        ],
        out_shape=[
            jax.ShapeDtypeStruct((N_PAD, D), jnp.float32),
            jax.ShapeDtypeStruct((N_PAD, D), jnp.float32),
        ],
    )(sp, dinv, b2, fc1w, fc1b)


def _tc_mlp_body(e0_ref, w2_ref, b2_ref, w3_ref, b3_ref, out_ref):
    e1 = _leaky(e0_ref[...])
    e2 = _leaky(jnp.dot(e1, w2_ref[...], preferred_element_type=jnp.float32)
                + b2_ref[...][None, :])
    out_ref[...] = (jnp.dot(e2, w3_ref[...], preferred_element_type=jnp.float32)
                    + b3_ref[...][None, :])


def _tc_mlp(e0, fc2w, fc2b, fc3w, fc3b):
    blk = 2560
    n_e = e0.shape[0]
    grid = n_e // blk
    return pl.pallas_call(
        _tc_mlp_body,
        grid=(grid,),
        in_specs=[
            pl.BlockSpec((blk, D), lambda i: (i, 0)),
            pl.BlockSpec((D, 64), lambda i: (0, 0)),
            pl.BlockSpec((64,), lambda i: (0,)),
            pl.BlockSpec((64, 3), lambda i: (0, 0)),
            pl.BlockSpec((3,), lambda i: (0,)),
        ],
        out_specs=pl.BlockSpec((blk, 3), lambda i: (i, 0)),
        out_shape=jax.ShapeDtypeStruct((n_e, 3), jnp.float32),
    )(e0, fc2w, fc2b, fc3w, fc3b)


def kernel(x, edge_index, W1, b1, W2, b2, fc1W, fc1b, fc2W, fc2b, fc3W, fc3b):
    src = edge_index[0].astype(jnp.int32)
    dst = edge_index[1].astype(jnp.int32)
    x_pad = jnp.zeros((N_PAD, D), jnp.float32).at[:N_NODES].set(x)
    zeros_n = jnp.zeros((N_PAD,), jnp.float32)
    zeros_nd = jnp.zeros((N_PAD, D), jnp.float32)

    degp = _sc_degree(dst, zeros_n)
    y1, dinv = _tc_pre(x_pad, W1, degp)

    sp1 = _sc_conv(y1, src, dst, zeros_nd)
    y2 = _tc_mid(sp1, dinv, b1, W2)

    sp2 = _sc_conv2(y2, src, dst, zeros_nd)
    a, b = _tc_post(sp2, dinv, b2, fc1W, fc1b)

    outs = []
    for _edge in _sc_edges:
        e0p = _edge(a, b, src, dst)
        outs.append(_tc_mlp(e0p, fc2W, fc2b, fc3W, fc3b))
    return jnp.concatenate(outs, axis=0)
